# baseline (speedup 1.0000x reference)
import functools

import jax
import jax.numpy as jnp
from jax import lax
from jax.experimental import pallas as pl
from jax.experimental.pallas import tpu as pltpu
from jax.experimental.pallas import tpu_sc as plsc

NUM_NODES = 1000000
H_DIM = 64
N_LOOKUPS = 819200

NC, NS = 2, 16
NW = NC * NS


TBLK = 128
N_FULL = NUM_NODES // TBLK
TAIL = NUM_NODES - N_FULL * TBLK
TFULL = N_FULL // NW
T_REM = N_FULL - TFULL * NW
TAIL_WID = T_REM


@functools.partial(
    pl.kernel,
    out_type=jax.ShapeDtypeStruct((NUM_NODES, 2 * H_DIM), jnp.float32),
    mesh=plsc.VectorSubcoreMesh(core_axis_name="c", subcore_axis_name="s"),
    scratch_types=[
        pltpu.VMEM((H_DIM, TBLK), jnp.float32),
        pltpu.VMEM((H_DIM, TBLK), jnp.float32),
        pltpu.VMEM((TBLK, 2 * H_DIM), jnp.float32),
        pltpu.VMEM((TBLK, 2 * H_DIM), jnp.float32),
        pltpu.VMEM((H_DIM * TAIL,), jnp.float32),
        pltpu.SemaphoreType.DMA,
        pltpu.SemaphoreType.DMA,
        pltpu.SemaphoreType.DMA,
        pltpu.SemaphoreType.DMA,
    ],
    compiler_params=pltpu.CompilerParams(needs_layout_passes=False),
)
def _transpose_kernel(wt_hbm, wtail_hbm, wp_hbm, i0, i1, o0, o1, ih,
                      gi0, gi1, so0, so1):
    wid = lax.axis_index("s") * NC + lax.axis_index("c")
    ib = (i0, i1)
    ob = (o0, o1)
    isem = (gi0, gi1)
    osem = (so0, so1)

    lane = lax.iota(jnp.int32, 16)
    rot = [lax.rem(lane + k, 16) for k in range(16)]
    colv = [lane + j0 for j0 in range(0, TBLK, 16)]

    def in_start(k, b):
        blk = wid + k * NW
        pltpu.async_copy(wt_hbm.at[:, pl.ds(blk * TBLK, TBLK)], ib[b],
                         isem[b])

    def in_wait(b):
        pltpu.make_async_copy(wt_hbm.at[:, pl.ds(0, TBLK)], ib[b],
                              isem[b]).wait()

    def out_start(k, b):
        blk = wid + k * NW
        pltpu.async_copy(ob[b], wp_hbm.at[pl.ds(blk * TBLK, TBLK)], osem[b])

    def out_wait(b):
        pltpu.make_async_copy(ob[b], wp_hbm.at[pl.ds(0, TBLK)],
                              osem[b]).wait()

    def transpose_ref(src, dst, njj):
        def kbody(k, carry):
            rotk = lax.rem(lane + k, 16)
            for jj in range(njj):
                for cc in range(H_DIM // 16):
                    rowv = rotk + (cc * 16)
                    vals = plsc.load_gather(src, [rowv, colv[jj]])
                    plsc.store_scatter(dst, [colv[jj], rowv], vals)
            return carry
        lax.fori_loop(0, 16, kbody, 0)

    def transpose(b):
        transpose_ref(ib[b], ob[b], TBLK // 16)

    in_start(0, 0)
    in_wait(0)
    in_start(1, 1)
    transpose(0)
    out_start(0, 0)
    in_wait(1)
    in_start(2, 0)
    transpose(1)
    out_start(1, 1)

    def body(jo, carry):
        for u in range(2):
            kk = jo * 2 + 2 + u
            in_wait(u)
            in_start(kk + 1, 1 - u)
            out_wait(u)
            transpose(u)
            out_start(kk, u)
        return carry

    lax.fori_loop(0, (TFULL - 4) // 2, body, 0)

    in_wait(0)
    in_start(TFULL - 1, 1)
    out_wait(0)
    transpose(0)
    out_start(TFULL - 2, 0)
    in_wait(1)
    out_wait(1)
    transpose(1)
    out_start(TFULL - 1, 1)

    @pl.when(wid < T_REM)
    def _extra():
        in_start(TFULL, 0)
        in_wait(0)
        out_wait(0)
        transpose(0)
        out_start(TFULL, 0)
        out_wait(0)

    @pl.when(wid == TAIL_WID)
    def _tail():
        pltpu.sync_copy(wtail_hbm, ih)
        out_wait(0)

        def tail_kbody(k, carry):
            rotk = lax.rem(lane + k, 16)
            for jj in range(TAIL // 16):
                for cc in range(H_DIM // 16):
                    rowv = rotk + (cc * 16)
                    flat = colv[jj] * H_DIM + rowv
                    vals = plsc.load_gather(ih, [flat])
                    plsc.store_scatter(ob[0], [colv[jj], rowv], vals)
            return carry
        lax.fori_loop(0, 16, tail_kbody, 0)
        pltpu.async_copy(ob[0].at[pl.ds(0, TAIL)],
                         wp_hbm.at[pl.ds(N_FULL * TBLK, TAIL)], osem[0])
        pltpu.make_async_copy(ob[0].at[pl.ds(0, TAIL)],
                              wp_hbm.at[pl.ds(N_FULL * TBLK, TAIL)],
                              osem[0]).wait()

    @pl.when(jnp.logical_and(wid >= T_REM, wid != TAIL_WID))
    def _drain0():
        out_wait(0)

    out_wait(1)


B_PER_W = N_LOOKUPS // NW
CHUNK = 256
N_CHUNKS = B_PER_W // CHUNK
NBUF = 3


@functools.partial(
    pl.kernel,
    out_type=jax.ShapeDtypeStruct((N_LOOKUPS, 2 * H_DIM), jnp.float32),
    mesh=plsc.VectorSubcoreMesh(core_axis_name="c", subcore_axis_name="s"),
    scratch_types=[
        pltpu.VMEM((B_PER_W,), jnp.int32),
        pltpu.VMEM((CHUNK, 2 * H_DIM), jnp.float32),
        pltpu.VMEM((CHUNK, 2 * H_DIM), jnp.float32),
        pltpu.VMEM((CHUNK, 2 * H_DIM), jnp.float32),
        pltpu.SemaphoreType.DMA,
        pltpu.SemaphoreType.DMA,
        pltpu.SemaphoreType.DMA,
        pltpu.SemaphoreType.DMA,
        pltpu.SemaphoreType.DMA,
        pltpu.SemaphoreType.DMA,
    ],
)
def _gather_kernel(idx_hbm, wp_hbm, out_hbm, idx_v, buf0, buf1, buf2,
                   g0, g1, g2, s0, s1, s2):
    wid = lax.axis_index("s") * NC + lax.axis_index("c")
    base = wid * B_PER_W
    bufs = (buf0, buf1, buf2)
    gsems = (g0, g1, g2)
    ssems = (s0, s1, s2)

    pltpu.sync_copy(idx_hbm.at[pl.ds(base, B_PER_W)], idx_v)

    def gather_start(i, b):
        pltpu.async_copy(wp_hbm.at[idx_v.at[pl.ds(i * CHUNK, CHUNK)]],
                         bufs[b], gsems[b])

    def gather_wait(b):
        pltpu.make_async_copy(wp_hbm.at[pl.ds(0, CHUNK)], bufs[b],
                              gsems[b]).wait()

    def store_start(i, b):
        pltpu.async_copy(bufs[b], out_hbm.at[pl.ds(base + i * CHUNK, CHUNK)],
                         ssems[b])

    def store_wait(b):
        pltpu.make_async_copy(bufs[b], out_hbm.at[pl.ds(0, CHUNK)],
                              ssems[b]).wait()

    gather_start(0, 0)
    gather_start(1, 1)

    gather_wait(0)
    store_start(0, 0)
    gather_start(2, 2)

    def body(jo, carry):
        for u in range(3):
            j = jo * 3 + 1 + u
            b = (1 + u) % NBUF
            bn = (3 + u) % NBUF
            gather_wait(b)
            store_start(j, b)
            store_wait(bn)
            gather_start(j + 2, bn)
        return carry

    lax.fori_loop(0, (N_CHUNKS - 4) // 3, body, 0)

    store_wait((N_CHUNKS - 1) % NBUF)
    gather_start(N_CHUNKS - 1, (N_CHUNKS - 1) % NBUF)
    for j in range(N_CHUNKS - 3, N_CHUNKS):
        b = j % NBUF
        gather_wait(b)
        store_start(j, b)
    for j in range(N_CHUNKS - 3, N_CHUNKS):
        store_wait(j % NBUF)


def kernel(node_id, weight):
    node_id = jnp.squeeze(node_id).astype(jnp.int32)
    wtail = weight[N_FULL * TBLK:, :].reshape(-1)
    wp = _transpose_kernel(weight.T, wtail)
    return _gather_kernel(node_id, wp)[:, :H_DIM]


# speedup vs baseline: 1.0021x; 1.0021x over previous
import functools

import jax
import jax.numpy as jnp
from jax import lax
from jax.experimental import pallas as pl
from jax.experimental.pallas import tpu as pltpu
from jax.experimental.pallas import tpu_sc as plsc

NUM_NODES = 1000000
H_DIM = 64
N_LOOKUPS = 819200

NC, NS = 2, 16
NW = NC * NS
B_PER_W = N_LOOKUPS // NW
CHUNK = 256
N_CHUNKS = B_PER_W // CHUNK
NBUF = 3


@functools.partial(
    pl.kernel,
    out_type=jax.ShapeDtypeStruct((N_LOOKUPS, 2 * H_DIM), jnp.float32),
    mesh=plsc.VectorSubcoreMesh(core_axis_name="c", subcore_axis_name="s"),
    scratch_types=[
        pltpu.VMEM((B_PER_W,), jnp.int32),
        pltpu.VMEM((CHUNK, 2 * H_DIM), jnp.float32),
        pltpu.VMEM((CHUNK, 2 * H_DIM), jnp.float32),
        pltpu.VMEM((CHUNK, 2 * H_DIM), jnp.float32),
        pltpu.SemaphoreType.DMA,
        pltpu.SemaphoreType.DMA,
        pltpu.SemaphoreType.DMA,
        pltpu.SemaphoreType.DMA,
        pltpu.SemaphoreType.DMA,
        pltpu.SemaphoreType.DMA,
    ],
)
def _gather_kernel(idx_hbm, wp_hbm, out_hbm, idx_v, buf0, buf1, buf2,
                   g0, g1, g2, s0, s1, s2):
    wid = lax.axis_index("s") * NC + lax.axis_index("c")
    base = wid * B_PER_W
    bufs = (buf0, buf1, buf2)
    gsems = (g0, g1, g2)
    ssems = (s0, s1, s2)

    pltpu.sync_copy(idx_hbm.at[pl.ds(base, B_PER_W)], idx_v)

    def gather_start(i, b):
        pltpu.async_copy(wp_hbm.at[idx_v.at[pl.ds(i * CHUNK, CHUNK)]],
                         bufs[b], gsems[b])

    def gather_wait(b):
        pltpu.make_async_copy(wp_hbm.at[pl.ds(0, CHUNK)], bufs[b],
                              gsems[b]).wait()

    def store_start(i, b):
        pltpu.async_copy(bufs[b], out_hbm.at[pl.ds(base + i * CHUNK, CHUNK)],
                         ssems[b])

    def store_wait(b):
        pltpu.make_async_copy(bufs[b], out_hbm.at[pl.ds(0, CHUNK)],
                              ssems[b]).wait()

    gather_start(0, 0)
    gather_start(1, 1)

    gather_wait(0)
    store_start(0, 0)
    gather_start(2, 2)

    def body(jo, carry):
        for u in range(3):
            j = jo * 3 + 1 + u
            b = (1 + u) % NBUF
            bn = (3 + u) % NBUF
            gather_wait(b)
            store_start(j, b)
            store_wait(bn)
            gather_start(j + 2, bn)
        return carry

    lax.fori_loop(0, (N_CHUNKS - 4) // 3, body, 0)

    store_wait((N_CHUNKS - 1) % NBUF)
    gather_start(N_CHUNKS - 1, (N_CHUNKS - 1) % NBUF)
    for j in range(N_CHUNKS - 3, N_CHUNKS):
        b = j % NBUF
        gather_wait(b)
        store_start(j, b)
    for j in range(N_CHUNKS - 3, N_CHUNKS):
        store_wait(j % NBUF)


def kernel(node_id, weight):
    node_id = jnp.squeeze(node_id).astype(jnp.int32)
    wp = jnp.concatenate(
        [weight, jnp.zeros((NUM_NODES, H_DIM), jnp.float32)], axis=1)
    return _gather_kernel(node_id, wp)[:, :H_DIM]
